# Initial kernel scaffold; baseline (speedup 1.0000x reference)
import jax
import jax.numpy as jnp
from jax.experimental import pallas as pl


def kernel(query, reference_points, value, W_off, b_off, W_attn, b_attn):
    raise NotImplementedError("write your pallas kernel here")


# speedup vs baseline: 5.5601x; 5.5601x over previous
import functools

import numpy as np
import jax
import jax.numpy as jnp
from jax import lax
from jax.experimental import pallas as pl
from jax.experimental.pallas import tpu as pltpu
from jax.experimental.pallas import tpu_sc as plsc

NUM_HEADS = 8
SUM_PTS = 16
HEAD_DIM = 32
SPATIAL = [(80, 80), (40, 40), (20, 20), (10, 10)]
LEN_V = sum(h * w for h, w in SPATIAL)

_LANE = np.arange(128)
_PT = _LANE % 16
_LVL = _PT // 4
_HH = _LANE // 16
_W_NP = np.array([w for (_, w) in SPATIAL], np.float32)[_LVL]
_H_NP = np.array([h for (h, _) in SPATIAL], np.float32)[_LVL]
_LVL_BASE = np.cumsum([0] + [h * w for (h, w) in SPATIAL])[:4]
_BASE_NP = (_LVL_BASE[_LVL] * NUM_HEADS + _HH).astype(np.int32)

_PERM = np.array([h * 32 + p * 2 + xy
                  for xy in (0, 1) for h in range(8) for p in range(16)])

_PERMC = np.empty(32, np.int64)
_PERMC[0::2] = np.arange(16)
_PERMC[1::2] = np.arange(16) + 16


def _prep_body(q_ref, rp_ref, wo_ref, bo_ref, wa_ref, ba_ref,
               wv_ref, hv_ref, wvi_ref, base_ref, idx_ref, wgt_ref):
    b = pl.program_id(0)
    q = q_ref[0]
    offs = jnp.dot(q, wo_ref[:], preferred_element_type=jnp.float32) + bo_ref[:]
    logits = jnp.dot(q, wa_ref[:], preferred_element_type=jnp.float32) + ba_ref[:]
    e = jnp.exp(logits)
    row_h = lax.broadcasted_iota(jnp.int32, (128, 128), 0) // SUM_PTS
    col_h = lax.broadcasted_iota(jnp.int32, (128, 128), 1) // SUM_PTS
    gm = (row_h == col_h).astype(jnp.float32)
    denom = jnp.dot(e, gm, preferred_element_type=jnp.float32)
    attnw = e / denom

    rp = rp_ref[0]
    cx = rp[:, 0:1]
    cy = rp[:, 1:2]
    rw = rp[:, 2:3]
    rh = rp[:, 3:4]
    locx = cx + offs[:, :128] * (rw * 0.125)
    locy = cy + offs[:, 128:] * (rh * 0.125)

    wv = wv_ref[:]
    hv = hv_ref[:]
    wvi = wvi_ref[:]
    base = base_ref[:] + b * (NUM_HEADS * LEN_V)
    nh = NUM_HEADS

    gx = locx * wv - 0.5
    gy = locy * hv - 0.5
    x0 = jnp.floor(gx)
    y0 = jnp.floor(gy)
    fx = gx - x0
    fy = gy - y0

    for c, (dx, dy) in enumerate([(0, 0), (1, 0), (0, 1), (1, 1)]):
        xc = x0 + dx
        yc = y0 + dy
        valid = ((xc >= 0) & (xc <= wv - 1) & (yc >= 0) & (yc <= hv - 1))
        wx = fx if dx else (1.0 - fx)
        wy = fy if dy else (1.0 - fy)
        wcorn = attnw * wx * wy * valid.astype(jnp.float32)
        xi = jnp.clip(xc, 0.0, wv - 1).astype(jnp.int32)
        yi = jnp.clip(yc, 0.0, hv - 1).astype(jnp.int32)
        idx = base + (yi * wvi + xi) * nh
        pair, half = c // 2, c % 2
        idx_ref[pair * 1000:(pair + 1) * 1000,
                half * 128:(half + 1) * 128] = idx
        wgt_ref[c * 1000:(c + 1) * 1000, :] = wcorn


_prep = pl.pallas_call(
    _prep_body,
    grid=(16,),
    in_specs=[
        pl.BlockSpec((1, 1000, 256), lambda b: (b, 0, 0)),
        pl.BlockSpec((1, 1000, 4), lambda b: (b, 0, 0)),
        pl.BlockSpec((256, 256), lambda b: (0, 0)),
        pl.BlockSpec((1, 256), lambda b: (0, 0)),
        pl.BlockSpec((256, 128), lambda b: (0, 0)),
        pl.BlockSpec((1, 128), lambda b: (0, 0)),
        pl.BlockSpec((1, 128), lambda b: (0, 0)),
        pl.BlockSpec((1, 128), lambda b: (0, 0)),
        pl.BlockSpec((1, 128), lambda b: (0, 0)),
        pl.BlockSpec((1, 128), lambda b: (0, 0)),
    ],
    out_specs=[
        pl.BlockSpec((2000, 256), lambda b: (b, 0)),
        pl.BlockSpec((4000, 128), lambda b: (b, 0)),
    ],
    out_shape=[
        jax.ShapeDtypeStruct((32000, 256), jnp.int32),
        jax.ShapeDtypeStruct((64000, 128), jnp.float32),
    ],
)


@functools.lru_cache(maxsize=None)
def _make_sc_gather():
    mesh = plsc.VectorSubcoreMesh(core_axis_name="c", subcore_axis_name="s")

    @functools.partial(
        pl.kernel,
        mesh=mesh,
        out_type=jax.ShapeDtypeStruct((4096000,), jnp.float32),
        scratch_types=[
            pltpu.VMEM((40, 256), jnp.int32),
            pltpu.VMEM((80, 128), jnp.float32),
            pltpu.VMEM((512, 32), jnp.bfloat16),
            pltpu.VMEM((512, 32), jnp.bfloat16),
            pltpu.VMEM((512, 32), jnp.bfloat16),
            pltpu.VMEM((512, 32), jnp.bfloat16),
            pltpu.VMEM((5120,), jnp.float32),
            pltpu.SemaphoreType.DMA,
            pltpu.SemaphoreType.DMA,
            pltpu.SemaphoreType.DMA,
            pltpu.SemaphoreType.DMA,
        ],
        compiler_params=pltpu.CompilerParams(use_tc_tiling_on_sc=False,
                                             needs_layout_passes=False),
    )
    def sc_k(table, idxr, wgtr, outr, idx_v, wgt_v, buf0, buf1, buf2, buf3,
             out_v, sem0, sem1, sem2, sem3):
        wid = lax.axis_index("s") * 2 + lax.axis_index("c")
        q0 = wid * 500
        bufs = [buf0, buf1, buf2, buf3]
        sems = [sem0, sem1, sem2, sem3]

        def issue(k, buf, sem):
            for pair in range(2):
                pltpu.async_copy(table.at[idx_v.at[pair * 20 + k]],
                                 buf.at[pl.ds(pair * 256, 256)], sem)

        def drain(buf, sem):
            pltpu.make_async_copy(table.at[pl.ds(0, 512)], buf, sem).wait()

        lanes = [jnp.full((16,), p, jnp.int32) for p in range(16)]

        def compute(k, buf):
            def hbody(h, carry3):
                acc0 = jnp.zeros((16,), jnp.float32)
                acc1 = jnp.zeros((16,), jnp.float32)
                for c in range(4):
                    w16 = wgt_v[c * 20 + k, pl.ds(h * 16, 16)]
                    for p in range(16):
                        j = c * 128 + h * 16 + p
                        w = w16.at[lanes[p]].get(mode="promise_in_bounds")
                        lo, hi = plsc.unpack(
                            buf[j, 0:32], format=plsc.PackFormat.INTERLEAVED)
                        acc0 = acc0 + w * lo
                        acc1 = acc1 + w * hi
                out_v[pl.ds(k * 256 + h * 32, 16)] = acc0
                out_v[pl.ds(k * 256 + h * 32 + 16, 16)] = acc1
                return carry3

            lax.fori_loop(0, 8, hbody, 0)

        def chunk(cc, carry):
            qc = q0 + cc * 20
            b = q0 // 1000
            qq = qc - b * 1000
            for pair in range(2):
                pltpu.sync_copy(
                    idxr.at[pl.ds(b * 2000 + pair * 1000 + qq, 20)],
                    idx_v.at[pl.ds(pair * 20, 20)])
            for c in range(4):
                pltpu.sync_copy(wgtr.at[pl.ds(b * 4000 + c * 1000 + qq, 20)],
                                wgt_v.at[pl.ds(c * 20, 20)])
            for k in range(3):
                issue(k, bufs[k], sems[k])

            def step(t, carry2):
                for kk in range(4):
                    k = 4 * t + kk
                    nxt = (kk + 3) % 4
                    if kk == 0:
                        issue(k + 3, bufs[nxt], sems[nxt])
                    else:
                        @pl.when(t < 4)
                        def _():
                            issue(k + 3, bufs[nxt], sems[nxt])
                    drain(bufs[kk], sems[kk])
                    compute(k, bufs[kk])
                return carry2

            lax.fori_loop(0, 5, step, 0)
            pltpu.sync_copy(out_v, outr.at[pl.ds(qc * 256, 5120)])
            return carry

        lax.fori_loop(0, 25, chunk, 0)

    return sc_k


def kernel(query, reference_points, value, W_off, b_off, W_attn, b_attn):
    bs, len_q, _ = query.shape
    wo = W_off[:, _PERM]
    bo = b_off[_PERM].reshape(1, 256)
    ba = b_attn.reshape(1, 128)
    rp = reference_points.reshape(bs, len_q, 4)
    wvc = jnp.asarray(_W_NP).reshape(1, 128)
    hvc = jnp.asarray(_H_NP).reshape(1, 128)
    wvic = jnp.asarray(_W_NP.astype(np.int32)).reshape(1, 128)
    basec = jnp.asarray(_BASE_NP).reshape(1, 128)
    idx, wgt = _prep(query, rp, wo, bo, W_attn, ba, wvc, hvc, wvic, basec)
    table = value[..., _PERMC].astype(jnp.bfloat16).reshape(
        bs * LEN_V * NUM_HEADS, HEAD_DIM)
    out = _make_sc_gather()(table, idx, wgt)
    return out.reshape(bs, len_q, 256)


# speedup vs baseline: 86.2532x; 15.5129x over previous
import functools

import numpy as np
import jax
import jax.numpy as jnp
from jax import lax
from jax.experimental import pallas as pl
from jax.experimental.pallas import tpu as pltpu
from jax.experimental.pallas import tpu_sc as plsc

NUM_HEADS = 8
SUM_PTS = 16
HEAD_DIM = 32
SPATIAL = [(80, 80), (40, 40), (20, 20), (10, 10)]
LEN_V = sum(h * w for h, w in SPATIAL)

_LANE = np.arange(128)
_PT = _LANE % 16
_LVL = _PT // 4
_HH = _LANE // 16
_W_NP = np.array([w for (_, w) in SPATIAL], np.float32)[_LVL]
_H_NP = np.array([h for (h, _) in SPATIAL], np.float32)[_LVL]
_LVL_BASE = np.cumsum([0] + [h * w for (h, w) in SPATIAL])[:4]
_BASE_NP = (_LVL_BASE[_LVL] * NUM_HEADS + _HH).astype(np.int32)

_PERM = np.array([h * 32 + p * 2 + xy
                  for xy in (0, 1) for h in range(8) for p in range(16)])

_PERMC = np.empty(32, np.int64)
_PERMC[0::2] = np.arange(16)
_PERMC[1::2] = np.arange(16) + 16


def _prep_body(q_ref, rp_ref, wo_ref, bo_ref, wa_ref, ba_ref,
               wv_ref, hv_ref, wvi_ref, base_ref, idx_ref, wgt_ref):
    b = pl.program_id(0)
    q = q_ref[0]
    offs = jnp.dot(q, wo_ref[:], preferred_element_type=jnp.float32) + bo_ref[:]
    logits = jnp.dot(q, wa_ref[:], preferred_element_type=jnp.float32) + ba_ref[:]
    e = jnp.exp(logits)
    row_h = lax.broadcasted_iota(jnp.int32, (128, 128), 0) // SUM_PTS
    col_h = lax.broadcasted_iota(jnp.int32, (128, 128), 1) // SUM_PTS
    gm = (row_h == col_h).astype(jnp.float32)
    denom = jnp.dot(e, gm, preferred_element_type=jnp.float32)
    attnw = e / denom

    rp = rp_ref[0]
    cx = rp[:, 0:1]
    cy = rp[:, 1:2]
    rw = rp[:, 2:3]
    rh = rp[:, 3:4]
    locx = cx + offs[:, :128] * (rw * 0.125)
    locy = cy + offs[:, 128:] * (rh * 0.125)

    wv = wv_ref[:]
    hv = hv_ref[:]
    wvi = wvi_ref[:]
    base = base_ref[:] + b * (NUM_HEADS * LEN_V)
    nh = NUM_HEADS

    gx = locx * wv - 0.5
    gy = locy * hv - 0.5
    x0 = jnp.floor(gx)
    y0 = jnp.floor(gy)
    fx = gx - x0
    fy = gy - y0

    for c, (dx, dy) in enumerate([(0, 0), (1, 0), (0, 1), (1, 1)]):
        xc = x0 + dx
        yc = y0 + dy
        valid = ((xc >= 0) & (xc <= wv - 1) & (yc >= 0) & (yc <= hv - 1))
        wx = fx if dx else (1.0 - fx)
        wy = fy if dy else (1.0 - fy)
        wcorn = attnw * wx * wy * valid.astype(jnp.float32)
        xi = jnp.clip(xc, 0.0, wv - 1).astype(jnp.int32)
        yi = jnp.clip(yc, 0.0, hv - 1).astype(jnp.int32)
        idx = base + (yi * wvi + xi) * nh
        pair, half = c // 2, c % 2
        idx_ref[pair * 1000:(pair + 1) * 1000,
                half * 128:(half + 1) * 128] = idx
        wgt_ref[c * 1000:(c + 1) * 1000, :] = wcorn


_prep = pl.pallas_call(
    _prep_body,
    grid=(16,),
    in_specs=[
        pl.BlockSpec((1, 1000, 256), lambda b: (b, 0, 0)),
        pl.BlockSpec((1, 1000, 4), lambda b: (b, 0, 0)),
        pl.BlockSpec((256, 256), lambda b: (0, 0)),
        pl.BlockSpec((1, 256), lambda b: (0, 0)),
        pl.BlockSpec((256, 128), lambda b: (0, 0)),
        pl.BlockSpec((1, 128), lambda b: (0, 0)),
        pl.BlockSpec((1, 128), lambda b: (0, 0)),
        pl.BlockSpec((1, 128), lambda b: (0, 0)),
        pl.BlockSpec((1, 128), lambda b: (0, 0)),
        pl.BlockSpec((1, 128), lambda b: (0, 0)),
    ],
    out_specs=[
        pl.BlockSpec((2000, 256), lambda b: (b, 0)),
        pl.BlockSpec((4000, 128), lambda b: (b, 0)),
    ],
    out_shape=[
        jax.ShapeDtypeStruct((32000, 256), jnp.int32),
        jax.ShapeDtypeStruct((64000, 128), jnp.float32),
    ],
)


@functools.lru_cache(maxsize=None)
def _make_sc_gather():
    mesh = plsc.VectorSubcoreMesh(core_axis_name="c", subcore_axis_name="s")

    @functools.partial(
        pl.kernel,
        mesh=mesh,
        out_type=jax.ShapeDtypeStruct((4096000,), jnp.float32),
        scratch_types=[
            pltpu.VMEM((40, 256), jnp.int32),
            pltpu.VMEM((80, 128), jnp.float32),
            pltpu.VMEM((512, 32), jnp.bfloat16),
            pltpu.VMEM((512, 32), jnp.bfloat16),
            pltpu.VMEM((512, 32), jnp.bfloat16),
            pltpu.VMEM((512, 32), jnp.bfloat16),
            pltpu.VMEM((5120,), jnp.float32),
            pltpu.SemaphoreType.DMA,
            pltpu.SemaphoreType.DMA,
            pltpu.SemaphoreType.DMA,
            pltpu.SemaphoreType.DMA,
        ],
        compiler_params=pltpu.CompilerParams(use_tc_tiling_on_sc=False,
                                             needs_layout_passes=False),
    )
    def sc_k(table, idxr, wgtr, outr, idx_v, wgt_v, buf0, buf1, buf2, buf3,
             out_v, sem0, sem1, sem2, sem3):
        wid = lax.axis_index("s") * 2 + lax.axis_index("c")
        q0 = wid * 500
        bufs = [buf0, buf1, buf2, buf3]
        sems = [sem0, sem1, sem2, sem3]

        def issue(k, buf, sem):
            for pair in range(2):
                pltpu.async_copy(table.at[idx_v.at[pair * 20 + k]],
                                 buf.at[pl.ds(pair * 256, 256)], sem)

        def drain(buf, sem):
            pltpu.make_async_copy(table.at[pl.ds(0, 512)], buf, sem).wait()

        lanes = [jnp.full((16,), p, jnp.int32) for p in range(16)]

        def compute(k, buf):
            evens = lax.iota(jnp.int32, 16) * 2

            def hbody(h, carry3):
                acc0 = jnp.zeros((16,), jnp.float32)
                acc1 = jnp.zeros((16,), jnp.float32)
                for c in range(4):
                    w16 = wgt_v[c * 20 + k, pl.ds(h * 16, 16)]
                    for p in range(16):
                        j = c * 128 + h * 16 + p
                        w = w16.at[lanes[p]].get(mode="promise_in_bounds")
                        ev, od = plsc.unpack(
                            buf[j, 0:32], format=plsc.PackFormat.INTERLEAVED)
                        acc0 = acc0 + w * ev
                        acc1 = acc1 + w * od
                base = k * 256 + h * 32 + evens
                plsc.store_scatter(out_v, [base], acc0)
                plsc.store_scatter(out_v, [base + 1], acc1)
                return carry3

            lax.fori_loop(0, 8, hbody, 0)

        def chunk(cc, carry):
            qc = q0 + cc * 20
            b = q0 // 1000
            qq = qc - b * 1000
            for pair in range(2):
                pltpu.sync_copy(
                    idxr.at[pl.ds(b * 2000 + pair * 1000 + qq, 20)],
                    idx_v.at[pl.ds(pair * 20, 20)])
            for c in range(4):
                pltpu.sync_copy(wgtr.at[pl.ds(b * 4000 + c * 1000 + qq, 20)],
                                wgt_v.at[pl.ds(c * 20, 20)])
            for k in range(3):
                issue(k, bufs[k], sems[k])

            def step(t, carry2):
                for kk in range(4):
                    k = 4 * t + kk
                    nxt = (kk + 3) % 4
                    if kk == 0:
                        issue(k + 3, bufs[nxt], sems[nxt])
                    else:
                        @pl.when(t < 4)
                        def _():
                            issue(k + 3, bufs[nxt], sems[nxt])
                    drain(bufs[kk], sems[kk])
                    compute(k, bufs[kk])
                return carry2

            lax.fori_loop(0, 5, step, 0)
            pltpu.sync_copy(out_v, outr.at[pl.ds(qc * 256, 5120)])
            return carry

        lax.fori_loop(0, 25, chunk, 0)

    return sc_k


def kernel(query, reference_points, value, W_off, b_off, W_attn, b_attn):
    bs, len_q, _ = query.shape
    wo = W_off[:, _PERM]
    bo = b_off[_PERM].reshape(1, 256)
    ba = b_attn.reshape(1, 128)
    rp = reference_points.reshape(bs, len_q, 4)
    wvc = jnp.asarray(_W_NP).reshape(1, 128)
    hvc = jnp.asarray(_H_NP).reshape(1, 128)
    wvic = jnp.asarray(_W_NP.astype(np.int32)).reshape(1, 128)
    basec = jnp.asarray(_BASE_NP).reshape(1, 128)
    idx, wgt = _prep(query, rp, wo, bo, W_attn, ba, wvc, hvc, wvic, basec)
    table = value.astype(jnp.bfloat16).reshape(bs * LEN_V * NUM_HEADS,
                                               HEAD_DIM)
    out = _make_sc_gather()(table, idx, wgt)
    return out.reshape(bs, len_q, 256)


# speedup vs baseline: 86.6875x; 1.0050x over previous
import functools

import numpy as np
import jax
import jax.numpy as jnp
from jax import lax
from jax.experimental import pallas as pl
from jax.experimental.pallas import tpu as pltpu
from jax.experimental.pallas import tpu_sc as plsc

NUM_HEADS = 8
SUM_PTS = 16
HEAD_DIM = 32
SPATIAL = [(80, 80), (40, 40), (20, 20), (10, 10)]
LEN_V = sum(h * w for h, w in SPATIAL)

_LANE = np.arange(128)
_PT = _LANE % 16
_LVL = _PT // 4
_HH = _LANE // 16
_W_NP = np.array([w for (_, w) in SPATIAL], np.float32)[_LVL]
_H_NP = np.array([h for (h, _) in SPATIAL], np.float32)[_LVL]
_LVL_BASE = np.cumsum([0] + [h * w for (h, w) in SPATIAL])[:4]
_BASE_NP = (_LVL_BASE[_LVL] * NUM_HEADS + _HH).astype(np.int32)

_PERM = np.array([h * 32 + p * 2 + xy
                  for xy in (0, 1) for h in range(8) for p in range(16)])


def _prep_body(q_ref, rp_ref, wo_ref, bo_ref, wa_ref, ba_ref,
               wv_ref, hv_ref, wvi_ref, base_ref, idx_ref, wgt_ref):
    b = pl.program_id(0)
    q = q_ref[0]
    offs = jnp.dot(q, wo_ref[:], preferred_element_type=jnp.float32) + bo_ref[:]
    logits = jnp.dot(q, wa_ref[:], preferred_element_type=jnp.float32) + ba_ref[:]
    e = jnp.exp(logits)
    row_h = lax.broadcasted_iota(jnp.int32, (128, 128), 0) // SUM_PTS
    col_h = lax.broadcasted_iota(jnp.int32, (128, 128), 1) // SUM_PTS
    gm = (row_h == col_h).astype(jnp.float32)
    denom = jnp.dot(e, gm, preferred_element_type=jnp.float32)
    attnw = e / denom

    rp = rp_ref[0]
    cx = rp[:, 0:1]
    cy = rp[:, 1:2]
    rw = rp[:, 2:3]
    rh = rp[:, 3:4]
    locx = cx + offs[:, :128] * (rw * 0.125)
    locy = cy + offs[:, 128:] * (rh * 0.125)

    wv = wv_ref[:]
    hv = hv_ref[:]
    wvi = wvi_ref[:]
    base = base_ref[:] + b * (NUM_HEADS * LEN_V)
    nh = NUM_HEADS

    gx = locx * wv - 0.5
    gy = locy * hv - 0.5
    x0 = jnp.floor(gx)
    y0 = jnp.floor(gy)
    fx = gx - x0
    fy = gy - y0

    for c, (dx, dy) in enumerate([(0, 0), (1, 0), (0, 1), (1, 1)]):
        xc = x0 + dx
        yc = y0 + dy
        valid = ((xc >= 0) & (xc <= wv - 1) & (yc >= 0) & (yc <= hv - 1))
        wx = fx if dx else (1.0 - fx)
        wy = fy if dy else (1.0 - fy)
        wcorn = attnw * wx * wy * valid.astype(jnp.float32)
        xi = jnp.clip(xc, 0.0, wv - 1).astype(jnp.int32)
        yi = jnp.clip(yc, 0.0, hv - 1).astype(jnp.int32)
        idx = base + (yi * wvi + xi) * nh
        pair, half = c // 2, c % 2
        idx_ref[pair * 1000:(pair + 1) * 1000,
                half * 128:(half + 1) * 128] = idx
        wgt_ref[c * 1000:(c + 1) * 1000, :] = wcorn


_prep = pl.pallas_call(
    _prep_body,
    grid=(16,),
    in_specs=[
        pl.BlockSpec((1, 1000, 256), lambda b: (b, 0, 0)),
        pl.BlockSpec((1, 1000, 4), lambda b: (b, 0, 0)),
        pl.BlockSpec((256, 256), lambda b: (0, 0)),
        pl.BlockSpec((1, 256), lambda b: (0, 0)),
        pl.BlockSpec((256, 128), lambda b: (0, 0)),
        pl.BlockSpec((1, 128), lambda b: (0, 0)),
        pl.BlockSpec((1, 128), lambda b: (0, 0)),
        pl.BlockSpec((1, 128), lambda b: (0, 0)),
        pl.BlockSpec((1, 128), lambda b: (0, 0)),
        pl.BlockSpec((1, 128), lambda b: (0, 0)),
    ],
    out_specs=[
        pl.BlockSpec((2000, 256), lambda b: (b, 0)),
        pl.BlockSpec((4000, 128), lambda b: (b, 0)),
    ],
    out_shape=[
        jax.ShapeDtypeStruct((32000, 256), jnp.int32),
        jax.ShapeDtypeStruct((64000, 128), jnp.float32),
    ],
)


@functools.lru_cache(maxsize=None)
def _make_sc_gather():
    mesh = plsc.VectorSubcoreMesh(core_axis_name="c", subcore_axis_name="s")

    @functools.partial(
        pl.kernel,
        mesh=mesh,
        out_type=jax.ShapeDtypeStruct((4096000,), jnp.float32),
        scratch_types=[
            pltpu.VMEM((40, 256), jnp.int32),
            pltpu.VMEM((80, 128), jnp.float32),
            pltpu.VMEM((512, 32), jnp.float32),
            pltpu.VMEM((512, 32), jnp.float32),
            pltpu.VMEM((512, 32), jnp.float32),
            pltpu.VMEM((512, 32), jnp.float32),
            pltpu.VMEM((5120,), jnp.float32),
            pltpu.SemaphoreType.DMA,
            pltpu.SemaphoreType.DMA,
            pltpu.SemaphoreType.DMA,
            pltpu.SemaphoreType.DMA,
        ],
        compiler_params=pltpu.CompilerParams(use_tc_tiling_on_sc=False),
    )
    def sc_k(table, idxr, wgtr, outr, idx_v, wgt_v, buf0, buf1, buf2, buf3,
             out_v, sem0, sem1, sem2, sem3):
        wid = lax.axis_index("s") * 2 + lax.axis_index("c")
        q0 = wid * 500
        bufs = [buf0, buf1, buf2, buf3]
        sems = [sem0, sem1, sem2, sem3]

        def issue(k, buf, sem):
            for pair in range(2):
                pltpu.async_copy(table.at[idx_v.at[pair * 20 + k]],
                                 buf.at[pl.ds(pair * 256, 256)], sem)

        def drain(buf, sem):
            pltpu.make_async_copy(table.at[pl.ds(0, 512)], buf, sem).wait()

        lanes = [jnp.full((16,), p, jnp.int32) for p in range(16)]

        def compute(k, buf):
            def hbody(h, carry3):
                acc0 = jnp.zeros((16,), jnp.float32)
                acc1 = jnp.zeros((16,), jnp.float32)
                for c in range(4):
                    w16 = wgt_v[c * 20 + k, pl.ds(h * 16, 16)]
                    for p in range(16):
                        j = c * 128 + h * 16 + p
                        w = w16.at[lanes[p]].get(mode="promise_in_bounds")
                        acc0 = acc0 + w * buf[j, 0:16]
                        acc1 = acc1 + w * buf[j, 16:32]
                out_v[pl.ds(k * 256 + h * 32, 16)] = acc0
                out_v[pl.ds(k * 256 + h * 32 + 16, 16)] = acc1
                return carry3

            lax.fori_loop(0, 8, hbody, 0)

        def chunk(cc, carry):
            qc = q0 + cc * 20
            b = q0 // 1000
            qq = qc - b * 1000
            for pair in range(2):
                pltpu.sync_copy(
                    idxr.at[pl.ds(b * 2000 + pair * 1000 + qq, 20)],
                    idx_v.at[pl.ds(pair * 20, 20)])
            for c in range(4):
                pltpu.sync_copy(wgtr.at[pl.ds(b * 4000 + c * 1000 + qq, 20)],
                                wgt_v.at[pl.ds(c * 20, 20)])
            for k in range(3):
                issue(k, bufs[k], sems[k])

            def step(t, carry2):
                for kk in range(4):
                    k = 4 * t + kk
                    nxt = (kk + 3) % 4
                    if kk == 0:
                        issue(k + 3, bufs[nxt], sems[nxt])
                    else:
                        @pl.when(t < 4)
                        def _():
                            issue(k + 3, bufs[nxt], sems[nxt])
                    drain(bufs[kk], sems[kk])
                    compute(k, bufs[kk])
                return carry2

            lax.fori_loop(0, 5, step, 0)
            pltpu.sync_copy(out_v, outr.at[pl.ds(qc * 256, 5120)])
            return carry

        lax.fori_loop(0, 25, chunk, 0)

    return sc_k


def kernel(query, reference_points, value, W_off, b_off, W_attn, b_attn):
    bs, len_q, _ = query.shape
    wo = W_off[:, _PERM]
    bo = b_off[_PERM].reshape(1, 256)
    ba = b_attn.reshape(1, 128)
    rp = reference_points.reshape(bs, len_q, 4)
    wvc = jnp.asarray(_W_NP).reshape(1, 128)
    hvc = jnp.asarray(_H_NP).reshape(1, 128)
    wvic = jnp.asarray(_W_NP.astype(np.int32)).reshape(1, 128)
    basec = jnp.asarray(_BASE_NP).reshape(1, 128)
    idx, wgt = _prep(query, rp, wo, bo, W_attn, ba, wvc, hvc, wvic, basec)
    table = value.reshape(bs * LEN_V * NUM_HEADS, HEAD_DIM)
    out = _make_sc_gather()(table, idx, wgt)
    return out.reshape(bs, len_q, 256)
